# baseline (speedup 1.0000x reference)
import jax
import jax.numpy as jnp
from jax.experimental import pallas as pl

E = 8
TOPK = 2
D = 1024
H = 512
T = 2048

BT = 256


def _dense_body(x_ref, w_ref, idx_ref, w1_ref, w2_ref, y_ref):
    e = pl.program_id(1)

    @pl.when(e == 0)
    def _():
        y_ref[...] = jnp.zeros_like(y_ref)

    xb = x_ref[...].astype(jnp.bfloat16)
    w1 = w1_ref[0].astype(jnp.bfloat16)
    w2 = w2_ref[0].astype(jnp.bfloat16)

    h = jax.lax.dot_general(
        xb, w1, (((1,), (1,)), ((), ())), preferred_element_type=jnp.float32
    )
    gate = h[:, :H]
    up = h[:, H:]
    a = gate * jax.lax.logistic(gate) * up
    out = jax.lax.dot_general(
        a.astype(jnp.bfloat16), w2, (((1,), (1,)), ((), ())),
        preferred_element_type=jnp.float32,
    )

    mask = idx_ref[...] == e
    we = jnp.sum(jnp.where(mask, w_ref[...], 0.0), axis=1)
    y_ref[...] += out * we[:, None]


def kernel(x, weights, indices, W1, W2):
    nt = T // BT
    grid = (nt, E)
    return pl.pallas_call(
        _dense_body,
        grid=grid,
        in_specs=[
            pl.BlockSpec((BT, D), lambda i, e: (i, 0)),
            pl.BlockSpec((BT, TOPK), lambda i, e: (i, 0)),
            pl.BlockSpec((BT, TOPK), lambda i, e: (i, 0)),
            pl.BlockSpec((1, 2 * H, D), lambda i, e: (e, 0, 0)),
            pl.BlockSpec((1, D, H), lambda i, e: (e, 0, 0)),
        ],
        out_specs=pl.BlockSpec((BT, D), lambda i, e: (i, 0)),
        out_shape=jax.ShapeDtypeStruct((T, D), jnp.float32),
    )(x, weights, indices.astype(jnp.int32), W1, W2)


# speedup vs baseline: 1.5165x; 1.5165x over previous
import jax
import jax.numpy as jnp
from jax.experimental import pallas as pl

E = 8
TOPK = 2
D = 1024
H = 512
T = 2048

CT = 256


def _dense_body(x_ref, w_ref, idx_ref, w1_ref, w2_ref, y_ref):
    e = pl.program_id(0)

    @pl.when(e == 0)
    def _():
        y_ref[...] = jnp.zeros_like(y_ref)

    w1 = w1_ref[0]
    w2 = w2_ref[0]

    def chunk(c, _):
        sl = pl.ds(c * CT, CT)
        xb = x_ref[sl, :]
        h = jax.lax.dot_general(
            xb, w1, (((1,), (1,)), ((), ())), preferred_element_type=jnp.float32
        )
        gate = h[:, :H]
        up = h[:, H:]
        a = gate * jax.lax.logistic(gate) * up
        out = jax.lax.dot_general(
            a, w2, (((1,), (1,)), ((), ())), preferred_element_type=jnp.float32
        )
        mask = idx_ref[sl, :] == e
        we = jnp.sum(jnp.where(mask, w_ref[sl, :], 0.0), axis=1)
        y_ref[sl, :] += out * we[:, None]
        return 0

    jax.lax.fori_loop(0, T // CT, chunk, 0)


def kernel(x, weights, indices, W1, W2):
    return pl.pallas_call(
        _dense_body,
        grid=(E,),
        in_specs=[
            pl.BlockSpec((T, D), lambda e: (0, 0)),
            pl.BlockSpec((T, TOPK), lambda e: (0, 0)),
            pl.BlockSpec((T, TOPK), lambda e: (0, 0)),
            pl.BlockSpec((1, 2 * H, D), lambda e: (e, 0, 0)),
            pl.BlockSpec((1, D, H), lambda e: (e, 0, 0)),
        ],
        out_specs=pl.BlockSpec((T, D), lambda e: (0, 0)),
        out_shape=jax.ShapeDtypeStruct((T, D), jnp.float32),
    )(x, weights, indices.astype(jnp.int32), W1, W2)
